# baseline (speedup 1.0000x reference)
import functools

import jax
import jax.numpy as jnp
from jax import lax
from jax.experimental import pallas as pl
from jax.experimental.pallas import tpu as pltpu
from jax.experimental.pallas import tpu_sc as plsc

H, W = 480, 640
HW = H * W
N_EV = 500000
FLOW_SCALING = 128.0

NUM_WORKERS = 32
BPW = 15616
NSUB = 2
SUB = BPW // NSUB
MAIN = NUM_WORKERS * BPW
TAIL = N_EV - MAIN

R_TBL = HW // 4
RPT = R_TBL // 16
NB = 5
RB = RPT // NB


def _sc_gather_body(xs_hbm, ys_hbm, flow_hbm, out_hbm,
                    xs_v, ys_v, ii_v, wi_v, rows_v, valx_v, valy_v,
                    st_v, bld_v, tbl_hbm, sem):
    sid = lax.axis_index("s")
    cid = lax.axis_index("c")
    wid = sid * 2 + cid
    base = wid * BPW
    lane = lax.iota(jnp.int32, 16)

    patt = ((lane >> 3) * 4 + (lane & 3)) + ((lane >> 2) & 1) * (RB * 4)
    rbase = lane >> 3
    colv = lane & 7

    def build(k, carry):
        row0 = sid * RPT + k * RB
        pltpu.sync_copy(flow_hbm.at[pl.ds(row0 * 4, RB * 4)],
                        st_v.at[pl.ds(0, RB * 4)])
        pltpu.sync_copy(flow_hbm.at[pl.ds(HW + row0 * 4, RB * 4)],
                        st_v.at[pl.ds(RB * 4, RB * 4)])

        def grp(g, c2):
            v = plsc.load_gather(st_v, [g * 8 + patt])
            plsc.store_scatter(bld_v, [g * 2 + rbase, colv], v)
            return c2

        lax.fori_loop(0, RB // 2, grp, 0)
        pltpu.sync_copy(bld_v, tbl_hbm.at[pl.ds(cid * R_TBL + row0, RB)])
        return carry

    lax.fori_loop(0, NB, build, 0)
    plsc.subcore_barrier()

    def do_chunk(off, xsr, ysr, iir, wir, rr, vxr, vyr, ngrp):
        n = ngrp * 16
        pltpu.sync_copy(xs_hbm.at[pl.ds(off, n)], xsr)
        pltpu.sync_copy(ys_hbm.at[pl.ds(off, n)], ysr)

        tb = cid * R_TBL

        def grp(g, c2):
            s = pl.ds(g * 16, 16)
            idx = xsr[s] + ysr[s] * W
            iir[s] = (idx >> 2) + tb
            wir[s] = idx & 3
            return c2

        lax.fori_loop(0, ngrp, grp, 0)
        pltpu.async_copy(tbl_hbm.at[iir], rr, sem).wait()

        def ext(g, c2):
            s = pl.ds(g * 16, 16)
            ev = g * 16 + lane
            w = wir[s]
            vxr[s] = plsc.load_gather(rr, [ev, w])
            vyr[s] = plsc.load_gather(rr, [ev, w + 4])
            return c2

        lax.fori_loop(0, ngrp, ext, 0)
        pltpu.sync_copy(vxr, out_hbm.at[pl.ds(off, n)])
        pltpu.sync_copy(vyr, out_hbm.at[pl.ds(N_EV + off, n)])

    def sub(j, carry):
        do_chunk(base + j * SUB, xs_v, ys_v, ii_v, wi_v, rows_v,
                 valx_v, valy_v, SUB // 16)
        return carry

    lax.fori_loop(0, NSUB, sub, 0)

    @pl.when(wid == NUM_WORKERS - 1)
    def _tail():
        t = TAIL
        do_chunk(MAIN,
                 xs_v.at[pl.ds(0, t)], ys_v.at[pl.ds(0, t)],
                 ii_v.at[pl.ds(0, t)], wi_v.at[pl.ds(0, t)],
                 rows_v.at[pl.ds(0, t)],
                 valx_v.at[pl.ds(0, t)], valy_v.at[pl.ds(0, t)],
                 t // 16)


@functools.lru_cache(maxsize=1)
def _sc_gather():
    return functools.partial(
        pl.kernel,
        out_type=jax.ShapeDtypeStruct((2 * N_EV,), jnp.float32),
        mesh=plsc.VectorSubcoreMesh(core_axis_name="c", subcore_axis_name="s"),
        compiler_params=pltpu.CompilerParams(
            needs_layout_passes=False, use_tc_tiling_on_sc=False),
        scratch_types=[
            pltpu.VMEM((SUB,), jnp.int32),
            pltpu.VMEM((SUB,), jnp.int32),
            pltpu.VMEM((SUB,), jnp.int32),
            pltpu.VMEM((SUB,), jnp.int32),
            pltpu.VMEM((SUB, 8), jnp.float32),
            pltpu.VMEM((SUB,), jnp.float32),
            pltpu.VMEM((SUB,), jnp.float32),
            pltpu.VMEM((RB * 8,), jnp.float32),
            pltpu.VMEM((RB, 8), jnp.float32),
            pltpu.HBM((2 * R_TBL, 8), jnp.float32),
            pltpu.SemaphoreType.DMA,
        ],
    )(_sc_gather_body)


def _warp_body(flow_ref, out_ref):
    f = flow_ref[...]
    xii = lax.broadcasted_iota(jnp.int32, (1, H, W), 2)
    yii = lax.broadcasted_iota(jnp.int32, (1, H, W), 1)
    xi = xii.astype(jnp.float32)
    yi = yii.astype(jnp.float32)
    gx = 2.0 * xi / (W - 1) - 1.0
    gy = 2.0 * yi / (H - 1) - 1.0
    px = (gx + 1.0) * (W - 1) / 2.0
    py = (gy + 1.0) * (H - 1) / 2.0
    x0 = jnp.floor(px)
    y0 = jnp.floor(py)
    wx = px - x0
    wy = py - y0
    x0i = jnp.clip(x0.astype(jnp.int32), 0, W - 1)
    x1i = jnp.clip(x0i + 1, 0, W - 1)
    y0i = jnp.clip(y0.astype(jnp.int32), 0, H - 1)
    y1i = jnp.clip(y0i + 1, 0, H - 1)
    fxm = jnp.concatenate([f[:, :, :1], f[:, :, :-1]], axis=2)
    fxp = jnp.concatenate([f[:, :, 1:], f[:, :, -1:]], axis=2)
    g0 = jnp.where(x0i == xii, f, fxm)
    g1 = jnp.where(x1i == xii, f, fxp)
    g0u = jnp.concatenate([g0[:, :1, :], g0[:, :-1, :]], axis=1)
    g0d = jnp.concatenate([g0[:, 1:, :], g0[:, -1:, :]], axis=1)
    g1u = jnp.concatenate([g1[:, :1, :], g1[:, :-1, :]], axis=1)
    g1d = jnp.concatenate([g1[:, 1:, :], g1[:, -1:, :]], axis=1)
    cy0 = y0i == yii
    cy1 = y1i == yii
    v00 = jnp.where(cy0, g0, g0u)
    v01 = jnp.where(cy0, g1, g1u)
    v10 = jnp.where(cy1, g0, g0d)
    v11 = jnp.where(cy1, g1, g1d)
    samp = (v00 * (1.0 - wy) * (1.0 - wx) + v01 * (1.0 - wy) * wx
            + v10 * wy * (1.0 - wx) + v11 * wy * wx)
    ind = jnp.concatenate([xi, yi], axis=0)
    warped = ind + samp * FLOW_SCALING
    out_ref[...] = warped - ind


_warp = pl.pallas_call(
    _warp_body,
    out_shape=jax.ShapeDtypeStruct((2, H, W), jnp.float32),
)


def kernel(flow, event_list, event_mask, dt_input, dt_gt):
    flow_flat = flow.reshape(2 * HW)
    xs = event_list[0, :, 1]
    ys = event_list[0, :, 2]
    planes = _sc_gather()(xs, ys, flow_flat)
    event_flow = planes.reshape(2, N_EV).T[None]
    accum = _warp(flow.reshape(2, H, W)).reshape(1, 2, H, W)
    return event_flow, accum


# speedup vs baseline: 1.2080x; 1.2080x over previous
import functools

import jax
import jax.numpy as jnp
from jax import lax
from jax.experimental import pallas as pl
from jax.experimental.pallas import tpu as pltpu
from jax.experimental.pallas import tpu_sc as plsc

H, W = 480, 640
HW = H * W
N_EV = 500000
FLOW_SCALING = 128.0

NUM_WORKERS = 32
BPW = 15616
MAIN = NUM_WORKERS * BPW
TAIL = N_EV - MAIN


def _sc_gather_body(xs_hbm, ys_hbm, flow_hbm, out_hbm,
                    xs_v, ys_v, iix_v, iiy_v, valx_v, valy_v, sem):
    wid = lax.axis_index("s") * 2 + lax.axis_index("c")
    base = wid * BPW

    def do_chunk(off, xsr, ysr, iixr, iiyr, vxr, vyr, ngrp):
        n = ngrp * 16
        pltpu.sync_copy(xs_hbm.at[pl.ds(off, n)], xsr)
        pltpu.sync_copy(ys_hbm.at[pl.ds(off, n)], ysr)

        def grp(g, c2):
            s = pl.ds(g * 16, 16)
            ii = xsr[s] + ysr[s] * W
            iixr[s] = ii
            iiyr[s] = ii + HW
            return c2

        lax.fori_loop(0, ngrp, grp, 0)
        cx = pltpu.async_copy(flow_hbm.at[iixr], vxr, sem)
        cy = pltpu.async_copy(flow_hbm.at[iiyr], vyr, sem)
        cx.wait()
        cy.wait()
        pltpu.sync_copy(vxr, out_hbm.at[pl.ds(off, n)])
        pltpu.sync_copy(vyr, out_hbm.at[pl.ds(N_EV + off, n)])

    do_chunk(base, xs_v, ys_v, iix_v, iiy_v, valx_v, valy_v, BPW // 16)

    @pl.when(wid == NUM_WORKERS - 1)
    def _tail():
        t = TAIL
        do_chunk(MAIN,
                 xs_v.at[pl.ds(0, t)], ys_v.at[pl.ds(0, t)],
                 iix_v.at[pl.ds(0, t)], iiy_v.at[pl.ds(0, t)],
                 valx_v.at[pl.ds(0, t)], valy_v.at[pl.ds(0, t)],
                 t // 16)


@functools.lru_cache(maxsize=1)
def _sc_gather():
    return functools.partial(
        pl.kernel,
        out_type=jax.ShapeDtypeStruct((2 * N_EV,), jnp.float32),
        mesh=plsc.VectorSubcoreMesh(core_axis_name="c", subcore_axis_name="s"),
        compiler_params=pltpu.CompilerParams(
            needs_layout_passes=False, use_tc_tiling_on_sc=False),
        scratch_types=[
            pltpu.VMEM((BPW,), jnp.int32),
            pltpu.VMEM((BPW,), jnp.int32),
            pltpu.VMEM((BPW,), jnp.int32),
            pltpu.VMEM((BPW,), jnp.int32),
            pltpu.VMEM((BPW,), jnp.float32),
            pltpu.VMEM((BPW,), jnp.float32),
            pltpu.SemaphoreType.DMA,
        ],
    )(_sc_gather_body)


def _warp_body(flow_ref, out_ref):
    f = flow_ref[...]
    xii = lax.broadcasted_iota(jnp.int32, (1, H, W), 2)
    yii = lax.broadcasted_iota(jnp.int32, (1, H, W), 1)
    xi = xii.astype(jnp.float32)
    yi = yii.astype(jnp.float32)
    gx = 2.0 * xi / (W - 1) - 1.0
    gy = 2.0 * yi / (H - 1) - 1.0
    px = (gx + 1.0) * (W - 1) / 2.0
    py = (gy + 1.0) * (H - 1) / 2.0
    x0 = jnp.floor(px)
    y0 = jnp.floor(py)
    wx = px - x0
    wy = py - y0
    x0i = jnp.clip(x0.astype(jnp.int32), 0, W - 1)
    x1i = jnp.clip(x0i + 1, 0, W - 1)
    y0i = jnp.clip(y0.astype(jnp.int32), 0, H - 1)
    y1i = jnp.clip(y0i + 1, 0, H - 1)
    fxm = jnp.concatenate([f[:, :, :1], f[:, :, :-1]], axis=2)
    fxp = jnp.concatenate([f[:, :, 1:], f[:, :, -1:]], axis=2)
    g0 = jnp.where(x0i == xii, f, fxm)
    g1 = jnp.where(x1i == xii, f, fxp)
    g0u = jnp.concatenate([g0[:, :1, :], g0[:, :-1, :]], axis=1)
    g0d = jnp.concatenate([g0[:, 1:, :], g0[:, -1:, :]], axis=1)
    g1u = jnp.concatenate([g1[:, :1, :], g1[:, :-1, :]], axis=1)
    g1d = jnp.concatenate([g1[:, 1:, :], g1[:, -1:, :]], axis=1)
    cy0 = y0i == yii
    cy1 = y1i == yii
    v00 = jnp.where(cy0, g0, g0u)
    v01 = jnp.where(cy0, g1, g1u)
    v10 = jnp.where(cy1, g0, g0d)
    v11 = jnp.where(cy1, g1, g1d)
    samp = (v00 * (1.0 - wy) * (1.0 - wx) + v01 * (1.0 - wy) * wx
            + v10 * wy * (1.0 - wx) + v11 * wy * wx)
    ind = jnp.concatenate([xi, yi], axis=0)
    warped = ind + samp * FLOW_SCALING
    out_ref[...] = warped - ind


_warp = pl.pallas_call(
    _warp_body,
    out_shape=jax.ShapeDtypeStruct((2, H, W), jnp.float32),
)


def kernel(flow, event_list, event_mask, dt_input, dt_gt):
    flow_flat = flow.reshape(2 * HW)
    xs = event_list[0, :, 1]
    ys = event_list[0, :, 2]
    planes = _sc_gather()(xs, ys, flow_flat)
    event_flow = planes.reshape(2, N_EV).T[None]
    accum = _warp(flow.reshape(2, H, W)).reshape(1, 2, H, W)
    return event_flow, accum


# speedup vs baseline: 1.2204x; 1.0102x over previous
import functools

import jax
import jax.numpy as jnp
from jax import lax
from jax.experimental import pallas as pl
from jax.experimental.pallas import tpu as pltpu
from jax.experimental.pallas import tpu_sc as plsc

H, W = 480, 640
HW = H * W
N_EV = 500000
FLOW_SCALING = 128.0

NUM_WORKERS = 32
BPW = 15616
MAIN = NUM_WORKERS * BPW
TAIL = N_EV - MAIN


HALF = BPW // 2


def _sc_gather_body(xs_hbm, ys_hbm, flow_hbm, out_hbm,
                    xs_v, ys_v, iix_v, iiy_v, valx_v, valy_v, sem0, sem1):
    wid = lax.axis_index("s") * 2 + lax.axis_index("c")
    base = wid * BPW

    def fill(lo, ngrp):
        def grp(g, c2):
            s = pl.ds((lo + g) * 16, 16)
            ii = xs_v[s] + ys_v[s] * W
            iix_v[s] = ii
            iiy_v[s] = ii + HW
            return c2

        lax.fori_loop(0, ngrp, grp, 0)

    pltpu.sync_copy(xs_hbm.at[pl.ds(base, BPW)], xs_v)
    pltpu.sync_copy(ys_hbm.at[pl.ds(base, BPW)], ys_v)

    fill(0, HALF // 16)
    h0 = pl.ds(0, HALF)
    cx0 = pltpu.async_copy(flow_hbm.at[iix_v.at[h0]], valx_v.at[h0], sem0)
    cy0 = pltpu.async_copy(flow_hbm.at[iiy_v.at[h0]], valy_v.at[h0], sem0)
    fill(HALF // 16, HALF // 16)
    h1 = pl.ds(HALF, HALF)
    cx1 = pltpu.async_copy(flow_hbm.at[iix_v.at[h1]], valx_v.at[h1], sem1)
    cy1 = pltpu.async_copy(flow_hbm.at[iiy_v.at[h1]], valy_v.at[h1], sem1)
    cx0.wait()
    cy0.wait()
    pltpu.sync_copy(valx_v.at[h0], out_hbm.at[pl.ds(base, HALF)])
    pltpu.sync_copy(valy_v.at[h0], out_hbm.at[pl.ds(N_EV + base, HALF)])
    cx1.wait()
    cy1.wait()
    pltpu.sync_copy(valx_v.at[h1], out_hbm.at[pl.ds(base + HALF, HALF)])
    pltpu.sync_copy(valy_v.at[h1], out_hbm.at[pl.ds(N_EV + base + HALF, HALF)])

    @pl.when(wid == NUM_WORKERS - 1)
    def _tail():
        t = TAIL
        tt = pl.ds(0, t)
        pltpu.sync_copy(xs_hbm.at[pl.ds(MAIN, t)], xs_v.at[tt])
        pltpu.sync_copy(ys_hbm.at[pl.ds(MAIN, t)], ys_v.at[tt])
        fill(0, t // 16)
        cx = pltpu.async_copy(flow_hbm.at[iix_v.at[tt]], valx_v.at[tt], sem0)
        cy = pltpu.async_copy(flow_hbm.at[iiy_v.at[tt]], valy_v.at[tt], sem0)
        cx.wait()
        cy.wait()
        pltpu.sync_copy(valx_v.at[tt], out_hbm.at[pl.ds(MAIN, t)])
        pltpu.sync_copy(valy_v.at[tt], out_hbm.at[pl.ds(N_EV + MAIN, t)])


@functools.lru_cache(maxsize=1)
def _sc_gather():
    return functools.partial(
        pl.kernel,
        out_type=jax.ShapeDtypeStruct((2 * N_EV,), jnp.float32),
        mesh=plsc.VectorSubcoreMesh(core_axis_name="c", subcore_axis_name="s"),
        compiler_params=pltpu.CompilerParams(
            needs_layout_passes=False, use_tc_tiling_on_sc=False),
        scratch_types=[
            pltpu.VMEM((BPW,), jnp.int32),
            pltpu.VMEM((BPW,), jnp.int32),
            pltpu.VMEM((BPW,), jnp.int32),
            pltpu.VMEM((BPW,), jnp.int32),
            pltpu.VMEM((BPW,), jnp.float32),
            pltpu.VMEM((BPW,), jnp.float32),
            pltpu.SemaphoreType.DMA,
            pltpu.SemaphoreType.DMA,
        ],
    )(_sc_gather_body)


def _warp_body(flow_ref, out_ref):
    f = flow_ref[...]
    xii = lax.broadcasted_iota(jnp.int32, (1, H, W), 2)
    yii = lax.broadcasted_iota(jnp.int32, (1, H, W), 1)
    xi = xii.astype(jnp.float32)
    yi = yii.astype(jnp.float32)
    gx = 2.0 * xi / (W - 1) - 1.0
    gy = 2.0 * yi / (H - 1) - 1.0
    px = (gx + 1.0) * (W - 1) / 2.0
    py = (gy + 1.0) * (H - 1) / 2.0
    x0 = jnp.floor(px)
    y0 = jnp.floor(py)
    wx = px - x0
    wy = py - y0
    x0i = jnp.clip(x0.astype(jnp.int32), 0, W - 1)
    x1i = jnp.clip(x0i + 1, 0, W - 1)
    y0i = jnp.clip(y0.astype(jnp.int32), 0, H - 1)
    y1i = jnp.clip(y0i + 1, 0, H - 1)
    fxm = jnp.concatenate([f[:, :, :1], f[:, :, :-1]], axis=2)
    fxp = jnp.concatenate([f[:, :, 1:], f[:, :, -1:]], axis=2)
    g0 = jnp.where(x0i == xii, f, fxm)
    g1 = jnp.where(x1i == xii, f, fxp)
    g0u = jnp.concatenate([g0[:, :1, :], g0[:, :-1, :]], axis=1)
    g0d = jnp.concatenate([g0[:, 1:, :], g0[:, -1:, :]], axis=1)
    g1u = jnp.concatenate([g1[:, :1, :], g1[:, :-1, :]], axis=1)
    g1d = jnp.concatenate([g1[:, 1:, :], g1[:, -1:, :]], axis=1)
    cy0 = y0i == yii
    cy1 = y1i == yii
    v00 = jnp.where(cy0, g0, g0u)
    v01 = jnp.where(cy0, g1, g1u)
    v10 = jnp.where(cy1, g0, g0d)
    v11 = jnp.where(cy1, g1, g1d)
    samp = (v00 * (1.0 - wy) * (1.0 - wx) + v01 * (1.0 - wy) * wx
            + v10 * wy * (1.0 - wx) + v11 * wy * wx)
    ind = jnp.concatenate([xi, yi], axis=0)
    warped = ind + samp * FLOW_SCALING
    out_ref[...] = warped - ind


_warp = pl.pallas_call(
    _warp_body,
    out_shape=jax.ShapeDtypeStruct((2, H, W), jnp.float32),
)


def kernel(flow, event_list, event_mask, dt_input, dt_gt):
    flow_flat = flow.reshape(2 * HW)
    xs = event_list[0, :, 1]
    ys = event_list[0, :, 2]
    planes = _sc_gather()(xs, ys, flow_flat)
    event_flow = planes.reshape(2, N_EV).T[None]
    accum = _warp(flow.reshape(2, H, W)).reshape(1, 2, H, W)
    return event_flow, accum
